# baseline (speedup 1.0000x reference)
import jax
import jax.numpy as jnp
from jax.experimental import pallas as pl
from jax.experimental.pallas import tpu as pltpu

_BR = 512
_BC = 128


def _body(x_ref, u_ref, o_ref, carry_ref):
    c = pl.program_id(1)

    @pl.when(c == 0)
    def _():
        carry_ref[...] = jnp.zeros_like(carry_ref)

    t = x_ref[...]
    cs = jax.lax.dot(
        t, u_ref[...],
        precision=jax.lax.Precision.DEFAULT,
        preferred_element_type=jnp.float32,
    )
    res = cs + carry_ref[:, 0:1]
    o_ref[...] = res
    carry_ref[...] = jnp.broadcast_to(res[:, -1:], carry_ref.shape)


@jax.jit
def kernel(x):
    R, C = x.shape
    u = jnp.triu(jnp.ones((_BC, _BC), jnp.float32))
    grid = (R // _BR, C // _BC)
    return pl.pallas_call(
        _body,
        grid=grid,
        in_specs=[
            pl.BlockSpec((_BR, _BC), lambda r, c: (r, c)),
            pl.BlockSpec((_BC, _BC), lambda r, c: (0, 0)),
        ],
        out_specs=pl.BlockSpec((_BR, _BC), lambda r, c: (r, c)),
        out_shape=jax.ShapeDtypeStruct((R, C), x.dtype),
        scratch_shapes=[pltpu.VMEM((_BR, 128), jnp.float32)],
        compiler_params=pltpu.CompilerParams(
            dimension_semantics=("parallel", "arbitrary"),
        ),
    )(x, u)


# speedup vs baseline: 1.6619x; 1.6619x over previous
import jax
import jax.numpy as jnp
from jax.experimental import pallas as pl
from jax.experimental.pallas import tpu as pltpu

_BR = 512
_BC = 256


def _body(x_ref, u_ref, o_ref, carry_ref):
    c = pl.program_id(1)

    @pl.when(c == 0)
    def _():
        carry_ref[...] = jnp.zeros_like(carry_ref)

    t = x_ref[...]
    cs = jax.lax.dot(
        t, u_ref[...],
        precision=jax.lax.Precision.DEFAULT,
        preferred_element_type=jnp.float32,
    )
    res = cs + carry_ref[:, 0:1]
    o_ref[...] = res
    carry_ref[...] = jnp.broadcast_to(res[:, -1:], carry_ref.shape)


@jax.jit
def kernel(x):
    R, C = x.shape
    u = jnp.triu(jnp.ones((_BC, _BC), jnp.float32))
    grid = (R // _BR, C // _BC)
    return pl.pallas_call(
        _body,
        grid=grid,
        in_specs=[
            pl.BlockSpec((_BR, _BC), lambda r, c: (r, c)),
            pl.BlockSpec((_BC, _BC), lambda r, c: (0, 0)),
        ],
        out_specs=pl.BlockSpec((_BR, _BC), lambda r, c: (r, c)),
        out_shape=jax.ShapeDtypeStruct((R, C), x.dtype),
        scratch_shapes=[pltpu.VMEM((_BR, 128), jnp.float32)],
        compiler_params=pltpu.CompilerParams(
            dimension_semantics=("parallel", "arbitrary"),
        ),
    )(x, u)


# speedup vs baseline: 2.3000x; 1.3839x over previous
import jax
import jax.numpy as jnp
from jax.experimental import pallas as pl
from jax.experimental.pallas import tpu as pltpu

_BR = 1024
_BC = 256


def _body(x_ref, u_ref, o_ref, carry_ref):
    c = pl.program_id(1)

    @pl.when(c == 0)
    def _():
        carry_ref[...] = jnp.zeros_like(carry_ref)

    t = x_ref[...]
    cs = jax.lax.dot(
        t, u_ref[...],
        precision=jax.lax.Precision.DEFAULT,
        preferred_element_type=jnp.float32,
    )
    res = cs + carry_ref[:, 0:1]
    o_ref[...] = res
    carry_ref[...] = jnp.broadcast_to(res[:, -1:], carry_ref.shape)


@jax.jit
def kernel(x):
    R, C = x.shape
    u = jnp.triu(jnp.ones((_BC, _BC), jnp.float32))
    grid = (R // _BR, C // _BC)
    return pl.pallas_call(
        _body,
        grid=grid,
        in_specs=[
            pl.BlockSpec((_BR, _BC), lambda r, c: (r, c)),
            pl.BlockSpec((_BC, _BC), lambda r, c: (0, 0)),
        ],
        out_specs=pl.BlockSpec((_BR, _BC), lambda r, c: (r, c)),
        out_shape=jax.ShapeDtypeStruct((R, C), x.dtype),
        scratch_shapes=[pltpu.VMEM((_BR, 128), jnp.float32)],
        compiler_params=pltpu.CompilerParams(
            dimension_semantics=("parallel", "arbitrary"),
        ),
    )(x, u)


# speedup vs baseline: 2.6940x; 1.1713x over previous
import jax
import jax.numpy as jnp
from jax.experimental import pallas as pl
from jax.experimental.pallas import tpu as pltpu

_BR = 1024
_BC = 512
_SUB = 128
_K = _BC // _SUB


def _body(x_ref, u_ref, o_ref, carry_ref):
    c = pl.program_id(0)

    @pl.when(c == 0)
    def _():
        carry_ref[...] = jnp.zeros_like(carry_ref)

    t = x_ref[...]
    u = u_ref[...]
    css = []
    for i in range(_K):
        sub = t[:, i * _SUB:(i + 1) * _SUB]
        css.append(
            jax.lax.dot(
                sub, u,
                precision=jax.lax.Precision.DEFAULT,
                preferred_element_type=jnp.float32,
            )
        )
    off = carry_ref[:, 0:1]
    offs = [off]
    for i in range(_K - 1):
        off = off + css[i][:, -1:]
        offs.append(off)
    for i in range(_K):
        o_ref[:, i * _SUB:(i + 1) * _SUB] = css[i] + offs[i]
    carry_ref[...] = jnp.broadcast_to(offs[-1] + css[-1][:, -1:], carry_ref.shape)


@jax.jit
def kernel(x):
    R, C = x.shape
    u = jnp.triu(jnp.ones((_SUB, _SUB), jnp.float32))
    grid = (C // _BC,)
    return pl.pallas_call(
        _body,
        grid=grid,
        in_specs=[
            pl.BlockSpec((_BR, _BC), lambda c: (0, c)),
            pl.BlockSpec((_SUB, _SUB), lambda c: (0, 0)),
        ],
        out_specs=pl.BlockSpec((_BR, _BC), lambda c: (0, c)),
        out_shape=jax.ShapeDtypeStruct((R, C), x.dtype),
        scratch_shapes=[pltpu.VMEM((_BR, 128), jnp.float32)],
        compiler_params=pltpu.CompilerParams(
            dimension_semantics=("arbitrary",),
        ),
    )(x, u)


# speedup vs baseline: 3.0812x; 1.1437x over previous
import jax
import jax.numpy as jnp
from jax.experimental import pallas as pl
from jax.experimental.pallas import tpu as pltpu

_BR = 1024
_BC = 1024
_SUB = 128
_K = _BC // _SUB


def _body(x_ref, u_ref, o_ref, carry_ref):
    c = pl.program_id(0)

    @pl.when(c == 0)
    def _():
        carry_ref[...] = jnp.zeros_like(carry_ref)

    t = x_ref[...]
    u = u_ref[...]
    css = []
    for i in range(_K):
        sub = t[:, i * _SUB:(i + 1) * _SUB]
        css.append(
            jax.lax.dot(
                sub, u,
                precision=jax.lax.Precision.DEFAULT,
                preferred_element_type=jnp.float32,
            )
        )
    off = carry_ref[:, 0:1]
    offs = [off]
    for i in range(_K - 1):
        off = off + css[i][:, -1:]
        offs.append(off)
    for i in range(_K):
        o_ref[:, i * _SUB:(i + 1) * _SUB] = css[i] + offs[i]
    carry_ref[...] = jnp.broadcast_to(offs[-1] + css[-1][:, -1:], carry_ref.shape)


@jax.jit
def kernel(x):
    R, C = x.shape
    u = jnp.triu(jnp.ones((_SUB, _SUB), jnp.float32))
    grid = (C // _BC,)
    return pl.pallas_call(
        _body,
        grid=grid,
        in_specs=[
            pl.BlockSpec((_BR, _BC), lambda c: (0, c)),
            pl.BlockSpec((_SUB, _SUB), lambda c: (0, 0)),
        ],
        out_specs=pl.BlockSpec((_BR, _BC), lambda c: (0, c)),
        out_shape=jax.ShapeDtypeStruct((R, C), x.dtype),
        scratch_shapes=[pltpu.VMEM((_BR, 128), jnp.float32)],
        compiler_params=pltpu.CompilerParams(
            dimension_semantics=("arbitrary",),
        ),
    )(x, u)


# speedup vs baseline: 3.1741x; 1.0302x over previous
import jax
import jax.numpy as jnp
from jax.experimental import pallas as pl
from jax.experimental.pallas import tpu as pltpu

_BR = 1024
_BC = 2048
_SUB = 128
_K = _BC // _SUB


def _body(x_ref, u_ref, o_ref, carry_ref):
    c = pl.program_id(0)

    @pl.when(c == 0)
    def _():
        carry_ref[...] = jnp.zeros_like(carry_ref)

    t = x_ref[...]
    u = u_ref[...]
    css = []
    for i in range(_K):
        sub = t[:, i * _SUB:(i + 1) * _SUB]
        css.append(
            jax.lax.dot(
                sub, u,
                precision=jax.lax.Precision.DEFAULT,
                preferred_element_type=jnp.float32,
            )
        )
    off = carry_ref[:, 0:1]
    offs = [off]
    for i in range(_K - 1):
        off = off + css[i][:, -1:]
        offs.append(off)
    for i in range(_K):
        o_ref[:, i * _SUB:(i + 1) * _SUB] = css[i] + offs[i]
    carry_ref[...] = jnp.broadcast_to(offs[-1] + css[-1][:, -1:], carry_ref.shape)


@jax.jit
def kernel(x):
    R, C = x.shape
    u = jnp.triu(jnp.ones((_SUB, _SUB), jnp.float32))
    grid = (C // _BC,)
    return pl.pallas_call(
        _body,
        grid=grid,
        in_specs=[
            pl.BlockSpec((_BR, _BC), lambda c: (0, c)),
            pl.BlockSpec((_SUB, _SUB), lambda c: (0, 0)),
        ],
        out_specs=pl.BlockSpec((_BR, _BC), lambda c: (0, c)),
        out_shape=jax.ShapeDtypeStruct((R, C), x.dtype),
        scratch_shapes=[pltpu.VMEM((_BR, 128), jnp.float32)],
        compiler_params=pltpu.CompilerParams(
            dimension_semantics=("arbitrary",),
        ),
    )(x, u)


# speedup vs baseline: 3.6250x; 1.1421x over previous
import jax
import jax.numpy as jnp
from jax.experimental import pallas as pl
from jax.experimental.pallas import tpu as pltpu

_BR = 1024
_BC = 2048
_SUB = 128
_K = _BC // _SUB


def _body(x_ref, m_ref, o_ref, carry_ref):
    c = pl.program_id(0)

    @pl.when(c == 0)
    def _():
        carry_ref[...] = jnp.zeros_like(carry_ref)

    t = x_ref[...]
    m = m_ref[...]
    off = carry_ref[...]
    for i in range(_K):
        sub = t[:, i * _SUB:(i + 1) * _SUB]
        r = jax.lax.dot(
            sub, m,
            precision=jax.lax.Precision.DEFAULT,
            preferred_element_type=jnp.float32,
        )
        o_ref[:, i * _SUB:(i + 1) * _SUB] = r[:, :_SUB] + off
        off = off + r[:, _SUB:]
    carry_ref[...] = off


@jax.jit
def kernel(x):
    R, C = x.shape
    u = jnp.triu(jnp.ones((_SUB, _SUB), jnp.float32))
    m = jnp.concatenate([u, jnp.ones((_SUB, _SUB), jnp.float32)], axis=1)
    grid = (C // _BC,)
    return pl.pallas_call(
        _body,
        grid=grid,
        in_specs=[
            pl.BlockSpec((_BR, _BC), lambda c: (0, c)),
            pl.BlockSpec((_SUB, 2 * _SUB), lambda c: (0, 0)),
        ],
        out_specs=pl.BlockSpec((_BR, _BC), lambda c: (0, c)),
        out_shape=jax.ShapeDtypeStruct((R, C), x.dtype),
        scratch_shapes=[pltpu.VMEM((_BR, _SUB), jnp.float32)],
        compiler_params=pltpu.CompilerParams(
            dimension_semantics=("arbitrary",),
        ),
    )(x, m)
